# baseline (speedup 1.0000x reference)
import functools

import jax
import jax.numpy as jnp
from jax import lax
from jax.experimental import pallas as pl
from jax.experimental.pallas import tpu as pltpu
from jax.experimental.pallas import tpu_sc as plsc

N_NODES = 10000
N_EDGES = 320000
D_FEAT = 128
HIDDEN = 64

NC = 2
NS = 16
NW = NC * NS
HALF = D_FEAT // NC
EPW = N_EDGES // NW
EPT = N_EDGES // NS
K = 80
KE = 80
DEPTH = 2
NCHUNK = EPW // K
NCHUNK_E = EPT // KE
STRIPE = 640
TAIL_STRIPE = N_NODES - (NS - 1) * STRIPE
DEGW = 16

_mesh = plsc.VectorSubcoreMesh(core_axis_name="c", subcore_axis_name="s")


@functools.partial(
    pl.kernel,
    out_type=(
        jax.ShapeDtypeStruct((NC, N_NODES, HALF), jnp.float32),
        jax.ShapeDtypeStruct((NC, N_NODES, DEGW), jnp.float32),
    ),
    mesh=_mesh,
    scratch_types=(
        pltpu.VMEM((NCHUNK_E, KE), jnp.int32),
        pltpu.VMEM((NCHUNK_E, KE), jnp.int32),
        tuple(pltpu.VMEM((KE, HALF), jnp.float32) for _ in range(DEPTH)),
        pltpu.VMEM((KE, DEGW), jnp.float32),
        pltpu.VMEM_SHARED((N_NODES, HALF), jnp.float32),
        pltpu.VMEM_SHARED((N_NODES, DEGW), jnp.float32),
        tuple(pltpu.SemaphoreType.DMA for _ in range(DEPTH)),
        tuple(pltpu.SemaphoreType.DMA for _ in range(DEPTH)),
        pltpu.SemaphoreType.DMA,
    ),
    compiler_params=pltpu.CompilerParams(use_tc_tiling_on_sc=False, needs_layout_passes=False),
)
def _encoder(x2_hbm, src_hbm, dst_hbm, zfeat_hbm, zdeg_hbm, ones_hbm,
             agg_out, deg_out,
             srcv, dstv, bufs, ones, agg_sh, deg_sh, gsem, ssem, semd):
    c = lax.axis_index("c")
    s = lax.axis_index("s")
    r0 = pl.multiple_of(s * STRIPE, STRIPE)

    @pl.when(s < NS - 1)
    def _zero_main():
        pltpu.sync_copy(zfeat_hbm.at[pl.ds(r0, STRIPE)],
                        agg_sh.at[pl.ds(r0, STRIPE)])
        pltpu.sync_copy(zdeg_hbm.at[pl.ds(r0, STRIPE)],
                        deg_sh.at[pl.ds(r0, STRIPE)])

    @pl.when(s == NS - 1)
    def _zero_tail():
        pltpu.sync_copy(zfeat_hbm.at[pl.ds((NS - 1) * STRIPE, TAIL_STRIPE)],
                        agg_sh.at[pl.ds((NS - 1) * STRIPE, TAIL_STRIPE)])
        pltpu.sync_copy(zdeg_hbm.at[pl.ds((NS - 1) * STRIPE, TAIL_STRIPE)],
                        deg_sh.at[pl.ds((NS - 1) * STRIPE, TAIL_STRIPE)])

    pltpu.sync_copy(ones_hbm, ones)
    pltpu.sync_copy(src_hbm.at[s], srcv)
    pltpu.sync_copy(dst_hbm.at[s], dstv)
    plsc.subcore_barrier()

    def _gather(ci, j):
        pltpu.async_copy(x2_hbm.at[c].at[srcv.at[ci]], bufs[j], gsem[j])

    def _gather_wait(ci, j):
        pltpu.make_async_copy(x2_hbm.at[c].at[srcv.at[ci]], bufs[j],
                              gsem[j]).wait()

    def _scatter(ci, j):
        pltpu.async_copy(bufs[j], agg_sh.at[dstv.at[ci]], ssem[j], add=True)

        @pl.when(lax.rem(ci, NC) == c)
        def _deg():
            pltpu.async_copy(ones, deg_sh.at[dstv.at[ci]], semd, add=True).wait()

    def _scatter_wait(ci, j):
        pltpu.make_async_copy(bufs[j], agg_sh.at[dstv.at[ci]], ssem[j]).wait()

    for j in range(DEPTH):
        _gather(j, j)

    @pl.loop(0, NCHUNK_E // DEPTH)
    def _chunk(i):
        for j in range(DEPTH):
            ci = i * DEPTH + j
            _gather_wait(ci, j)
            _scatter(ci, j)
            _scatter_wait(ci, j)

            @pl.when(ci + DEPTH < NCHUNK_E)
            def _pref():
                _gather(ci + DEPTH, j)

    plsc.subcore_barrier()

    @pl.when(s < NS - 1)
    def _out_main():
        pltpu.sync_copy(agg_sh.at[pl.ds(r0, STRIPE)],
                        agg_out.at[c, pl.ds(r0, STRIPE)])
        pltpu.sync_copy(deg_sh.at[pl.ds(r0, STRIPE)],
                        deg_out.at[c, pl.ds(r0, STRIPE)])

    @pl.when(s == NS - 1)
    def _out_tail():
        pltpu.sync_copy(agg_sh.at[pl.ds((NS - 1) * STRIPE, TAIL_STRIPE)],
                        agg_out.at[c, pl.ds((NS - 1) * STRIPE, TAIL_STRIPE)])
        pltpu.sync_copy(deg_sh.at[pl.ds((NS - 1) * STRIPE, TAIL_STRIPE)],
                        deg_out.at[c, pl.ds((NS - 1) * STRIPE, TAIL_STRIPE)])


SWEEP = 1600
NGROUP = SWEEP // 16
EPC = N_EDGES // NC
NSWEEP = EPC // SWEEP
CH = 80
WLCAP = SWEEP + CH + 16
OWN = STRIPE
TRASH = OWN
AGG_ROWS = OWN + 8


@functools.partial(
    pl.kernel,
    out_type=(
        jax.ShapeDtypeStruct((NC, N_NODES * D_FEAT), jnp.float32),
        jax.ShapeDtypeStruct((NC, N_NODES), jnp.float32),
    ),
    mesh=_mesh,
    scratch_types=(
        tuple(pltpu.VMEM((SWEEP,), jnp.int32) for _ in range(2)),
        tuple(pltpu.VMEM((SWEEP,), jnp.int32) for _ in range(2)),
        tuple(pltpu.VMEM((WLCAP,), jnp.int32) for _ in range(2)),
        tuple(pltpu.VMEM((WLCAP,), jnp.int32) for _ in range(2)),
        tuple(pltpu.VMEM((CH, D_FEAT), jnp.float32) for _ in range(2)),
        pltpu.VMEM((AGG_ROWS * D_FEAT,), jnp.float32),
        pltpu.VMEM((AGG_ROWS,), jnp.float32),
        tuple(pltpu.SemaphoreType.DMA for _ in range(2)),
        tuple(pltpu.SemaphoreType.DMA for _ in range(2)),
        pltpu.SemaphoreType.DMA,
    ),
    compiler_params=pltpu.CompilerParams(use_tc_tiling_on_sc=False,
                                         needs_layout_passes=False),
)
def _encoder3(x_hbm, src_hbm, dst_hbm, zagg_hbm, zdeg_hbm, agg_out, deg_out,
              ssrc, sdst, wsrc, wdst, rb, agg_l, deg_l, isem, gsem, msem):
    c = lax.axis_index("c")
    s = lax.axis_index("s")
    lo = s * OWN
    base_e = c * EPC
    lane = lax.iota(jnp.int32, 16)
    lane0 = lane < 1
    ones16 = jnp.ones((16,), jnp.float32)
    zer16i = jnp.zeros((16,), jnp.int32)

    pltpu.sync_copy(zagg_hbm, agg_l)
    pltpu.sync_copy(zdeg_hbm, deg_l)

    def _idx_fetch(sw, p):
        off = pl.multiple_of(base_e + sw * SWEEP, 8)
        pltpu.async_copy(src_hbm.at[pl.ds(off, SWEEP)], ssrc[p], isem[p])
        pltpu.async_copy(dst_hbm.at[pl.ds(off, SWEEP)], sdst[p], isem[p])

    def _idx_wait(sw, p):
        off = pl.multiple_of(base_e + sw * SWEEP, 8)
        pltpu.make_async_copy(src_hbm.at[pl.ds(off, SWEEP)], ssrc[p],
                              isem[p]).wait()
        pltpu.make_async_copy(dst_hbm.at[pl.ds(off, SWEEP)], sdst[p],
                              isem[p]).wait()

    def _scan(p):
        @pl.loop(0, NGROUP, init_carry=zer16i, unroll=4)
        def cnt_final(g, cnt):
            sl = pl.ds(g * 16, 16)
            d16 = sdst[p][sl]
            t16 = d16 - lo
            m = t16.astype(jnp.uint32) < jnp.uint32(OWN)
            cs = plsc.cumsum(jnp.where(m, 1, 0))
            pos = cnt + cs - 1
            plsc.store_scatter(wsrc[p], [pos], ssrc[p][sl], mask=m)
            plsc.store_scatter(wdst[p], [pos], t16, mask=m)
            return cnt + plsc.all_reduce_population_count(m)

        cnt = cnt_final
        for g in range(CH // 16):
            pos = cnt + g * 16 + lane
            plsc.store_scatter(wsrc[p], [pos], zer16i)
            plsc.store_scatter(wdst[p], [pos], jnp.full((16,), TRASH, jnp.int32))
        n_scalar = jnp.max(cnt)
        return (n_scalar + (CH - 1)) // CH

    def _gchunk(p, t, b):
        pltpu.async_copy(x_hbm.at[wsrc[p].at[pl.ds(t * CH, CH)]],
                         rb[b], gsem[b])

    def _gwait(p, t, b):
        pltpu.make_async_copy(x_hbm.at[wsrc[p].at[pl.ds(t * CH, CH)]],
                              rb[b], gsem[b]).wait()

    def _accum(p, t, b):
        ebase = t * CH

        @pl.loop(0, CH, unroll=8)
        def _edge(j):
            dv = plsc.load_gather(wdst[p], [jnp.full((16,), ebase + j,
                                                     jnp.int32)])
            dbase = dv * D_FEAT
            for fg in range(D_FEAT // 16):
                v = rb[b][j, pl.ds(fg * 16, 16)]
                plsc.addupdate_scatter(agg_l, [dbase + (fg * 16) + lane], v)
            plsc.addupdate_scatter(deg_l, [dv], ones16, mask=lane0)

    def _drain(p, nch):
        _gchunk(p, 0, 0)

        @pl.loop(0, (nch + 1) // 2)
        def _pair(tt):
            t0 = tt * 2
            t1 = t0 + 1

            @pl.when(t1 < nch)
            def _():
                _gchunk(p, t1, 1)

            _gwait(p, t0, 0)
            _accum(p, t0, 0)

            @pl.when(t1 + 1 < nch)
            def _():
                _gchunk(p, t1 + 1, 0)

            @pl.when(t1 < nch)
            def _():
                _gwait(p, t1, 1)
                _accum(p, t1, 1)

    _idx_fetch(0, 0)

    @pl.loop(0, NSWEEP // 2)
    def _sweeps(i):
        sw0 = i * 2
        _idx_wait(sw0, 0)
        _idx_fetch(sw0 + 1, 1)
        n0 = _scan(0)
        _drain(0, n0)
        _idx_wait(sw0 + 1, 1)

        @pl.when(sw0 + 2 < NSWEEP)
        def _():
            _idx_fetch(sw0 + 2, 0)

        n1 = _scan(1)
        _drain(1, n1)

    @pl.when(s < NS - 1)
    def _out_main():
        pltpu.sync_copy(agg_l.at[pl.ds(0, OWN * D_FEAT)],
                        agg_out.at[c, pl.ds(lo * D_FEAT, OWN * D_FEAT)])
        pltpu.sync_copy(deg_l.at[pl.ds(0, OWN)],
                        deg_out.at[c, pl.ds(lo, OWN)])

    @pl.when(s == NS - 1)
    def _out_tail():
        pltpu.sync_copy(agg_l.at[pl.ds(0, TAIL_STRIPE * D_FEAT)],
                        agg_out.at[c, pl.ds((NS - 1) * STRIPE * D_FEAT,
                                            TAIL_STRIPE * D_FEAT)])
        pltpu.sync_copy(deg_l.at[pl.ds(0, TAIL_STRIPE)],
                        deg_out.at[c, pl.ds((NS - 1) * STRIPE, TAIL_STRIPE)])


def _dense_body(pa_ref, dg_ref, wmu_ref, bmu_ref, wlv_ref, blv_ref, eps_ref,
                mu_ref, lv_ref, z_ref):
    p = pa_ref[0] + pa_ref[1]
    deg = dg_ref[0] + dg_ref[1]
    agg = p / jnp.maximum(deg, 1.0)
    mu = jnp.dot(agg, wmu_ref[...], preferred_element_type=jnp.float32) + bmu_ref[...]
    lv = jnp.dot(agg, wlv_ref[...], preferred_element_type=jnp.float32) + blv_ref[...]
    z = mu + eps_ref[...] * jnp.exp(0.5 * lv)
    mu_ref[...] = mu
    lv_ref[...] = lv
    z_ref[...] = z


_DENSE_BLK = 1000


def _dense(pa, dg, W_mu, b_mu, W_lv, b_lv, eps):
    n_blocks = N_NODES // _DENSE_BLK
    return pl.pallas_call(
        _dense_body,
        grid=(n_blocks,),
        in_specs=[
            pl.BlockSpec((NC, _DENSE_BLK, D_FEAT), lambda i: (0, i, 0)),
            pl.BlockSpec((NC, _DENSE_BLK, 1), lambda i: (0, i, 0)),
            pl.BlockSpec((D_FEAT, HIDDEN), lambda i: (0, 0)),
            pl.BlockSpec((1, HIDDEN), lambda i: (0, 0)),
            pl.BlockSpec((D_FEAT, HIDDEN), lambda i: (0, 0)),
            pl.BlockSpec((1, HIDDEN), lambda i: (0, 0)),
            pl.BlockSpec((_DENSE_BLK, HIDDEN), lambda i: (i, 0)),
        ],
        out_specs=[
            pl.BlockSpec((_DENSE_BLK, HIDDEN), lambda i: (i, 0)),
            pl.BlockSpec((_DENSE_BLK, HIDDEN), lambda i: (i, 0)),
            pl.BlockSpec((_DENSE_BLK, HIDDEN), lambda i: (i, 0)),
        ],
        out_shape=[
            jax.ShapeDtypeStruct((N_NODES, HIDDEN), jnp.float32),
            jax.ShapeDtypeStruct((N_NODES, HIDDEN), jnp.float32),
            jax.ShapeDtypeStruct((N_NODES, HIDDEN), jnp.float32),
        ],
    )(pa, dg, W_mu, b_mu, W_lv, b_lv, eps)


@functools.partial(
    pl.kernel,
    out_type=jax.ShapeDtypeStruct((N_EDGES,), jnp.float32),
    mesh=_mesh,
    scratch_types=(
        pltpu.VMEM((NCHUNK, K), jnp.int32),
        pltpu.VMEM((NCHUNK, K), jnp.int32),
        pltpu.VMEM((K, HIDDEN), jnp.float32),
        pltpu.VMEM((K, HIDDEN), jnp.float32),
        pltpu.VMEM((K, HIDDEN), jnp.float32),
        pltpu.VMEM((K, HIDDEN), jnp.float32),
        pltpu.VMEM((EPW,), jnp.float32),
        pltpu.SemaphoreType.DMA,
        pltpu.SemaphoreType.DMA,
    ),
    compiler_params=pltpu.CompilerParams(use_tc_tiling_on_sc=False, needs_layout_passes=False),
)
def _decoder(z_hbm, src_hbm, dst_hbm, recon_out,
             srcv, dstv, zs0, zs1, zd0, zd1, outv, sem0, sem1):
    c = lax.axis_index("c")
    s = lax.axis_index("s")
    w = s * NC + c
    base = pl.multiple_of(w * EPW, 8)
    pltpu.sync_copy(src_hbm.at[w], srcv)
    pltpu.sync_copy(dst_hbm.at[w], dstv)

    def _gather(ci, zsb, zdb, sem):
        pltpu.async_copy(z_hbm.at[srcv.at[ci]], zsb, sem)
        pltpu.async_copy(z_hbm.at[dstv.at[ci]], zdb, sem)

    def _wait(ci, zsb, zdb, sem):
        pltpu.make_async_copy(z_hbm.at[srcv.at[ci]], zsb, sem).wait()
        pltpu.make_async_copy(z_hbm.at[dstv.at[ci]], zdb, sem).wait()

    def _compute(ci, zsb, zdb):
        for g in range(K // 16):
            row = lax.iota(jnp.int32, 16) + (g * 16)
            acc = jnp.zeros((16,), jnp.float32)
            for dd in range(HIDDEN):
                col = jnp.full((16,), dd, jnp.int32)
                a = plsc.load_gather(zsb, [row, col])
                b = plsc.load_gather(zdb, [row, col])
                acc = acc + a * b
            outv[pl.ds(ci * K + g * 16, 16)] = 1.0 / (1.0 + jnp.exp(-acc))

    _gather(0, zs0, zd0, sem0)

    @pl.loop(0, (NCHUNK - 1) // 2)
    def _chunk(i):
        ci0 = i * 2
        ci1 = ci0 + 1
        _gather(ci1, zs1, zd1, sem1)
        _wait(ci0, zs0, zd0, sem0)
        _compute(ci0, zs0, zd0)
        _gather(ci0 + 2, zs0, zd0, sem0)
        _wait(ci1, zs1, zd1, sem1)
        _compute(ci1, zs1, zd1)

    _wait(NCHUNK - 1, zs0, zd0, sem0)
    _compute(NCHUNK - 1, zs0, zd0)
    pltpu.sync_copy(outv, recon_out.at[pl.ds(base, EPW)])


def kernel(x, edge_index, W_mu, b_mu, W_lv, b_lv):
    ei = edge_index.astype(jnp.int32)
    src_d = ei[0].reshape(NW, NCHUNK, K)
    dst_d = ei[1].reshape(NW, NCHUNK, K)
    zagg = jnp.zeros((AGG_ROWS * D_FEAT,), jnp.float32)
    zdeg = jnp.zeros((AGG_ROWS,), jnp.float32)
    pa, dg = _encoder3(x, ei[0], ei[1], zagg, zdeg)
    pa = pa.reshape(NC, N_NODES, D_FEAT)
    dg = dg.reshape(NC, N_NODES, 1)
    eps = jax.random.normal(jax.random.key(42), (N_NODES, HIDDEN), jnp.float32)
    mu, lv, z = _dense(pa, dg, W_mu, b_mu.reshape(1, HIDDEN),
                       W_lv, b_lv.reshape(1, HIDDEN), eps)
    recon = _decoder(z, src_d, dst_d)
    return (recon, mu, lv, z)


# speedup vs baseline: 8.0016x; 8.0016x over previous
import functools

import jax
import jax.numpy as jnp
from jax import lax
from jax.experimental import pallas as pl
from jax.experimental.pallas import tpu as pltpu
from jax.experimental.pallas import tpu_sc as plsc

N_NODES = 10000
N_EDGES = 320000
D_FEAT = 128
HIDDEN = 64

NC = 2
NS = 16
NW = NC * NS
HALF = D_FEAT // NC
EPW = N_EDGES // NW
EPT = N_EDGES // NS
K = 80
KE = 80
DEPTH = 2
NCHUNK = EPW // K
NCHUNK_E = EPT // KE
STRIPE = 640
TAIL_STRIPE = N_NODES - (NS - 1) * STRIPE
DEGW = 16

_mesh = plsc.VectorSubcoreMesh(core_axis_name="c", subcore_axis_name="s")


@functools.partial(
    pl.kernel,
    out_type=(
        jax.ShapeDtypeStruct((NC, N_NODES, HALF), jnp.float32),
        jax.ShapeDtypeStruct((NC, N_NODES, DEGW), jnp.float32),
    ),
    mesh=_mesh,
    scratch_types=(
        pltpu.VMEM((NCHUNK_E, KE), jnp.int32),
        pltpu.VMEM((NCHUNK_E, KE), jnp.int32),
        tuple(pltpu.VMEM((KE, HALF), jnp.float32) for _ in range(DEPTH)),
        pltpu.VMEM((KE, DEGW), jnp.float32),
        pltpu.VMEM_SHARED((N_NODES, HALF), jnp.float32),
        pltpu.VMEM_SHARED((N_NODES, DEGW), jnp.float32),
        tuple(pltpu.SemaphoreType.DMA for _ in range(DEPTH)),
        tuple(pltpu.SemaphoreType.DMA for _ in range(DEPTH)),
        pltpu.SemaphoreType.DMA,
    ),
    compiler_params=pltpu.CompilerParams(use_tc_tiling_on_sc=False, needs_layout_passes=False),
)
def _encoder(x2_hbm, src_hbm, dst_hbm, zfeat_hbm, zdeg_hbm, ones_hbm,
             agg_out, deg_out,
             srcv, dstv, bufs, ones, agg_sh, deg_sh, gsem, ssem, semd):
    c = lax.axis_index("c")
    s = lax.axis_index("s")
    r0 = pl.multiple_of(s * STRIPE, STRIPE)

    @pl.when(s < NS - 1)
    def _zero_main():
        pltpu.sync_copy(zfeat_hbm.at[pl.ds(r0, STRIPE)],
                        agg_sh.at[pl.ds(r0, STRIPE)])
        pltpu.sync_copy(zdeg_hbm.at[pl.ds(r0, STRIPE)],
                        deg_sh.at[pl.ds(r0, STRIPE)])

    @pl.when(s == NS - 1)
    def _zero_tail():
        pltpu.sync_copy(zfeat_hbm.at[pl.ds((NS - 1) * STRIPE, TAIL_STRIPE)],
                        agg_sh.at[pl.ds((NS - 1) * STRIPE, TAIL_STRIPE)])
        pltpu.sync_copy(zdeg_hbm.at[pl.ds((NS - 1) * STRIPE, TAIL_STRIPE)],
                        deg_sh.at[pl.ds((NS - 1) * STRIPE, TAIL_STRIPE)])

    pltpu.sync_copy(ones_hbm, ones)
    pltpu.sync_copy(src_hbm.at[s], srcv)
    pltpu.sync_copy(dst_hbm.at[s], dstv)
    plsc.subcore_barrier()

    def _gather(ci, j):
        pltpu.async_copy(x2_hbm.at[c].at[srcv.at[ci]], bufs[j], gsem[j])

    def _gather_wait(ci, j):
        pltpu.make_async_copy(x2_hbm.at[c].at[srcv.at[ci]], bufs[j],
                              gsem[j]).wait()

    def _scatter(ci, j):
        pltpu.async_copy(bufs[j], agg_sh.at[dstv.at[ci]], ssem[j], add=True)

        @pl.when(lax.rem(ci, NC) == c)
        def _deg():
            pltpu.async_copy(ones, deg_sh.at[dstv.at[ci]], semd, add=True).wait()

    def _scatter_wait(ci, j):
        pltpu.make_async_copy(bufs[j], agg_sh.at[dstv.at[ci]], ssem[j]).wait()

    for j in range(DEPTH):
        _gather(j, j)

    @pl.loop(0, NCHUNK_E // DEPTH)
    def _chunk(i):
        for j in range(DEPTH):
            ci = i * DEPTH + j
            _gather_wait(ci, j)
            _scatter(ci, j)
            _scatter_wait(ci, j)

            @pl.when(ci + DEPTH < NCHUNK_E)
            def _pref():
                _gather(ci + DEPTH, j)

    plsc.subcore_barrier()

    @pl.when(s < NS - 1)
    def _out_main():
        pltpu.sync_copy(agg_sh.at[pl.ds(r0, STRIPE)],
                        agg_out.at[c, pl.ds(r0, STRIPE)])
        pltpu.sync_copy(deg_sh.at[pl.ds(r0, STRIPE)],
                        deg_out.at[c, pl.ds(r0, STRIPE)])

    @pl.when(s == NS - 1)
    def _out_tail():
        pltpu.sync_copy(agg_sh.at[pl.ds((NS - 1) * STRIPE, TAIL_STRIPE)],
                        agg_out.at[c, pl.ds((NS - 1) * STRIPE, TAIL_STRIPE)])
        pltpu.sync_copy(deg_sh.at[pl.ds((NS - 1) * STRIPE, TAIL_STRIPE)],
                        deg_out.at[c, pl.ds((NS - 1) * STRIPE, TAIL_STRIPE)])


def _dense_body(pa_ref, dg_ref, wmu_ref, bmu_ref, wlv_ref, blv_ref, eps_ref,
                mu_ref, lv_ref, z_ref):
    p = jnp.concatenate([pa_ref[0], pa_ref[1]], axis=-1)
    deg = (dg_ref[0] + dg_ref[1])[:, 0:1]
    agg = p / jnp.maximum(deg, 1.0)
    mu = jnp.dot(agg, wmu_ref[...], preferred_element_type=jnp.float32) + bmu_ref[...]
    lv = jnp.dot(agg, wlv_ref[...], preferred_element_type=jnp.float32) + blv_ref[...]
    z = mu + eps_ref[...] * jnp.exp(0.5 * lv)
    mu_ref[...] = mu
    lv_ref[...] = lv
    z_ref[...] = z


_DENSE_BLK = 1000


def _dense(pa, dg, W_mu, b_mu, W_lv, b_lv, eps):
    n_blocks = N_NODES // _DENSE_BLK
    return pl.pallas_call(
        _dense_body,
        grid=(n_blocks,),
        in_specs=[
            pl.BlockSpec((NC, _DENSE_BLK, HALF), lambda i: (0, i, 0)),
            pl.BlockSpec((NC, _DENSE_BLK, DEGW), lambda i: (0, i, 0)),
            pl.BlockSpec((D_FEAT, HIDDEN), lambda i: (0, 0)),
            pl.BlockSpec((1, HIDDEN), lambda i: (0, 0)),
            pl.BlockSpec((D_FEAT, HIDDEN), lambda i: (0, 0)),
            pl.BlockSpec((1, HIDDEN), lambda i: (0, 0)),
            pl.BlockSpec((_DENSE_BLK, HIDDEN), lambda i: (i, 0)),
        ],
        out_specs=[
            pl.BlockSpec((_DENSE_BLK, HIDDEN), lambda i: (i, 0)),
            pl.BlockSpec((_DENSE_BLK, HIDDEN), lambda i: (i, 0)),
            pl.BlockSpec((_DENSE_BLK, HIDDEN), lambda i: (i, 0)),
        ],
        out_shape=[
            jax.ShapeDtypeStruct((N_NODES, HIDDEN), jnp.float32),
            jax.ShapeDtypeStruct((N_NODES, HIDDEN), jnp.float32),
            jax.ShapeDtypeStruct((N_NODES, HIDDEN), jnp.float32),
        ],
    )(pa, dg, W_mu, b_mu, W_lv, b_lv, eps)


@functools.partial(
    pl.kernel,
    out_type=jax.ShapeDtypeStruct((N_EDGES,), jnp.float32),
    mesh=_mesh,
    scratch_types=(
        pltpu.VMEM((NCHUNK, K), jnp.int32),
        pltpu.VMEM((NCHUNK, K), jnp.int32),
        pltpu.VMEM((K, HIDDEN), jnp.float32),
        pltpu.VMEM((K, HIDDEN), jnp.float32),
        pltpu.VMEM((K, HIDDEN), jnp.float32),
        pltpu.VMEM((K, HIDDEN), jnp.float32),
        pltpu.VMEM((EPW,), jnp.float32),
        pltpu.SemaphoreType.DMA,
        pltpu.SemaphoreType.DMA,
    ),
    compiler_params=pltpu.CompilerParams(use_tc_tiling_on_sc=False, needs_layout_passes=False),
)
def _decoder(z_hbm, src_hbm, dst_hbm, recon_out,
             srcv, dstv, zs0, zs1, zd0, zd1, outv, sem0, sem1):
    c = lax.axis_index("c")
    s = lax.axis_index("s")
    w = s * NC + c
    base = pl.multiple_of(w * EPW, 8)
    pltpu.sync_copy(src_hbm.at[w], srcv)
    pltpu.sync_copy(dst_hbm.at[w], dstv)

    def _gather(ci, zsb, zdb, sem):
        pltpu.async_copy(z_hbm.at[srcv.at[ci]], zsb, sem)
        pltpu.async_copy(z_hbm.at[dstv.at[ci]], zdb, sem)

    def _wait(ci, zsb, zdb, sem):
        pltpu.make_async_copy(z_hbm.at[srcv.at[ci]], zsb, sem).wait()
        pltpu.make_async_copy(z_hbm.at[dstv.at[ci]], zdb, sem).wait()

    def _compute(ci, zsb, zdb):
        for g in range(K // 16):
            row = lax.iota(jnp.int32, 16) + (g * 16)
            acc = jnp.zeros((16,), jnp.float32)
            for dd in range(HIDDEN):
                col = jnp.full((16,), dd, jnp.int32)
                a = plsc.load_gather(zsb, [row, col])
                b = plsc.load_gather(zdb, [row, col])
                acc = acc + a * b
            outv[pl.ds(ci * K + g * 16, 16)] = 1.0 / (1.0 + jnp.exp(-acc))

    _gather(0, zs0, zd0, sem0)

    @pl.loop(0, (NCHUNK - 1) // 2)
    def _chunk(i):
        ci0 = i * 2
        ci1 = ci0 + 1
        _gather(ci1, zs1, zd1, sem1)
        _wait(ci0, zs0, zd0, sem0)
        _compute(ci0, zs0, zd0)
        _gather(ci0 + 2, zs0, zd0, sem0)
        _wait(ci1, zs1, zd1, sem1)
        _compute(ci1, zs1, zd1)

    _wait(NCHUNK - 1, zs0, zd0, sem0)
    _compute(NCHUNK - 1, zs0, zd0)
    pltpu.sync_copy(outv, recon_out.at[pl.ds(base, EPW)])


def kernel(x, edge_index, W_mu, b_mu, W_lv, b_lv):
    ei = edge_index.astype(jnp.int32)
    src_e = ei[0].reshape(NS, NCHUNK_E, KE)
    dst_e = ei[1].reshape(NS, NCHUNK_E, KE)
    src_d = ei[0].reshape(NW, NCHUNK, K)
    dst_d = ei[1].reshape(NW, NCHUNK, K)
    x2 = x.reshape(N_NODES, NC, HALF).transpose(1, 0, 2)
    zfeat = jnp.zeros((N_NODES, HALF), jnp.float32)
    zdeg = jnp.zeros((N_NODES, DEGW), jnp.float32)
    ones = jnp.ones((KE, DEGW), jnp.float32)
    pa, dg = _encoder(x2, src_e, dst_e, zfeat, zdeg, ones)
    eps = jax.random.normal(jax.random.key(42), (N_NODES, HIDDEN), jnp.float32)
    mu, lv, z = _dense(pa, dg, W_mu, b_mu.reshape(1, HIDDEN),
                       W_lv, b_lv.reshape(1, HIDDEN), eps)
    recon = _decoder(z, src_d, dst_d)
    return (recon, mu, lv, z)
